# Initial kernel scaffold; baseline (speedup 1.0000x reference)
import jax
import jax.numpy as jnp
from jax.experimental import pallas as pl


def kernel(src1, src2, memoryMartix):
    raise NotImplementedError("write your pallas kernel here")


# speedup vs baseline: 26.4089x; 26.4089x over previous
import jax
import jax.numpy as jnp
from jax.experimental import pallas as pl

TOPK = 20
N = 4096
BR = 256


def _block(m_ref, src2t_ref, src1t_ref, out_ref):
    a = m_ref[...]
    rowmax = jnp.max(a, axis=1, keepdims=True)
    e = jnp.exp(a - rowmax)
    denom = jnp.sum(e, axis=1, keepdims=True)
    t = rowmax
    for _ in range(TOPK - 1):
        t = jnp.max(jnp.where(a < t, a, -jnp.inf), axis=1, keepdims=True)
    w = jnp.where(a >= t, e / denom, 0.0)
    out_ref[...] = (
        jnp.dot(w, src2t_ref[...], preferred_element_type=jnp.float32)
        + src1t_ref[...]
    )


def kernel(src1, src2, memoryMartix):
    B, n, d = src1.shape
    bd = B * d
    src2t = src2.transpose(1, 0, 2).reshape(n, bd)
    src1t = src1.transpose(1, 0, 2).reshape(n, bd)
    out_t = pl.pallas_call(
        _block,
        grid=(n // BR,),
        in_specs=[
            pl.BlockSpec((BR, n), lambda i: (i, 0)),
            pl.BlockSpec((n, bd), lambda i: (0, 0)),
            pl.BlockSpec((BR, bd), lambda i: (i, 0)),
        ],
        out_specs=pl.BlockSpec((BR, bd), lambda i: (i, 0)),
        out_shape=jax.ShapeDtypeStruct((n, bd), jnp.float32),
    )(memoryMartix, src2t, src1t)
    return out_t.reshape(n, B, d).transpose(1, 0, 2)
